# baseline (speedup 1.0000x reference)
import functools

import jax
import jax.numpy as jnp
from jax import lax
from jax.experimental import pallas as pl
from jax.experimental.pallas import tpu as pltpu
from jax.experimental.pallas import tpu_sc as plsc

N = 10000
E = 320000
D = 128
H = 128
G = 64

NC = 2
NS = 16
NW = NC * NS
L = 16

CH = 80
EPT = E // NW
EPS = E // NS
ZR = 80
NZC = N // ZR
KE = 3
KD = 10


def _rsqrt16(y):
    i = lax.bitcast_convert_type(y, jnp.int32)
    i = jnp.int32(0x5F3759DF) - (i >> 1)
    r = lax.bitcast_convert_type(i, jnp.float32)
    half = y * 0.5
    for _ in range(3):
        r = r * (1.5 - half * r * r)
    return r


def _sc_body(src_hbm, dst_hbm, ew_hbm, x_hbm, pp_hbm, dinv_hbm,
             p_acc, deg_acc, dinvv, rows,
             isrc, idst, ewb, wbuf, idstd, ewbd,
             sl0, sl1, sg0, sg1, ss0, ss1):
    sem_l = (sl0, sl1)
    sem_g = (sg0, sg1)
    sem_s = (ss0, ss1)
    c = lax.axis_index("c")
    s = lax.axis_index("s")
    wid = c * NS + s

    zv = jnp.zeros((L,), jnp.float32)

    def _zero_rows(k, carry):
        rows[0, k // 8, pl.ds((k % 8) * L, L)] = zv
        return carry

    lax.fori_loop(0, ZR * 8, _zero_rows, 0)
    for j in range(CH // L):
        ewbd[0, pl.ds(j * L, L)] = zv

    descs = []
    for j in range((NZC + NS - 1) // NS):
        k = s + NS * j
        r0 = pl.multiple_of(k * ZR, 8)
        if (j + 1) * NS <= NZC:
            descs.append(pltpu.async_copy(
                rows.at[0], p_acc.at[pl.ds(r0, ZR)], sem_g[0]))
            descs.append(pltpu.async_copy(
                ewbd.at[0], deg_acc.at[pl.ds(r0, ZR)], sem_l[0]))
        else:
            @pl.when(k < NZC)
            def _():
                pltpu.sync_copy(rows.at[0], p_acc.at[pl.ds(r0, ZR)])
                pltpu.sync_copy(ewbd.at[0], deg_acc.at[pl.ds(r0, ZR)])
    for d in descs:
        d.wait()

    plsc.subcore_barrier()

    def _deg_group(g, carry):
        base = pl.multiple_of(s * EPS + g * (KD * CH), 8)
        descs = []
        for k in range(KD):
            b = base + k * CH
            descs.append(pltpu.async_copy(
                dst_hbm.at[pl.ds(b, CH)], idstd.at[k], sem_l[0]))
            descs.append(pltpu.async_copy(
                ew_hbm.at[pl.ds(b, CH)], ewbd.at[k], sem_l[0]))
        for d in descs:
            d.wait()
        descs = []
        for k in range(KD):
            descs.append(pltpu.async_copy(
                ewbd.at[k], deg_acc.at[idstd.at[k]], sem_s[0], add=True))
        for d in descs:
            d.wait()
        return carry

    lax.fori_loop(0, EPS // (KD * CH), _deg_group, 0)

    plsc.subcore_barrier()

    pltpu.sync_copy(deg_acc, dinvv)

    def _dinv(j, carry):
        y = dinvv[pl.ds(j * L, L)] + 1.0
        dinvv[pl.ds(j * L, L)] = _rsqrt16(y)
        return carry

    lax.fori_loop(0, N // L, _dinv, 0)

    @pl.when(jnp.logical_and(c == 0, s == 0))
    def _():
        pltpu.sync_copy(dinvv, dinv_hbm)

    NCH = EPT // CH

    def _q_base(q):
        return pl.multiple_of(wid * EPT + q * CH, 8)

    def _issue_loads(q, b6, par):
        b = _q_base(q)
        pltpu.async_copy(src_hbm.at[pl.ds(b, CH)], isrc.at[b6], sem_l[par])
        pltpu.async_copy(dst_hbm.at[pl.ds(b, CH)], idst.at[b6], sem_l[par])
        pltpu.async_copy(ew_hbm.at[pl.ds(b, CH)], ewb.at[b6], sem_l[par])

    def _wait_loads(b6, par):
        z = pl.ds(0, CH)
        pltpu.make_async_copy(src_hbm.at[z], isrc.at[b6], sem_l[par]).wait()
        pltpu.make_async_copy(dst_hbm.at[z], idst.at[b6], sem_l[par]).wait()
        pltpu.make_async_copy(ew_hbm.at[z], ewb.at[b6], sem_l[par]).wait()

    def _compute_w(b6):
        for j in range(CH // L):
            sv = isrc[b6, pl.ds(j * L, L)]
            dv = idst[b6, pl.ds(j * L, L)]
            dis = plsc.load_gather(dinvv, [sv])
            did = plsc.load_gather(dinvv, [dv])
            wbuf[pl.ds(b6 * CH + j * L, L)] = (
                ewb[b6, pl.ds(j * L, L)] * dis * did)

    def _wait_rows(b3, sem):
        pltpu.make_async_copy(x_hbm.at[pl.ds(0, CH)], rows.at[b3], sem).wait()

    def _chunk(g, b6, has_p1=True, has_p2=True, wait_sc=True):
        q1b6, q2b6 = (b6 + 1) % 6, (b6 + 2) % 6
        b3, q1b3 = b6 % 3, (b6 + 1) % 3
        par, q1par = b6 % 2, (b6 + 1) % 2
        if has_p1:
            _wait_loads(q1b6, q1par)
            _compute_w(q1b6)
            if wait_sc:
                _wait_rows((b6 + 2) % 3, sem_s[q1par])
            pltpu.async_copy(x_hbm.at[isrc.at[q1b6]], rows.at[q1b3],
                             sem_g[q1par])
        if has_p2:
            _issue_loads(6 * g + b6 + 2, q2b6, par)
        _wait_rows(b3, sem_g[par])

        @plsc.parallel_loop(b6 * CH, (b6 + 1) * CH, 1, unroll=8)
        def _scale(e):
            wk = plsc.load_gather(wbuf, [jnp.full((L,), e, jnp.int32)])
            for jj in range(D // L):
                rows[b3, e - b6 * CH, pl.ds(jj * L, L)] = (
                    rows[b3, e - b6 * CH, pl.ds(jj * L, L)] * wk)

        pltpu.async_copy(rows.at[b3], p_acc.at[idst.at[b6]], sem_s[par],
                         add=True)

    _issue_loads(0, 0, 0)
    _issue_loads(1, 1, 1)
    _wait_loads(0, 0)
    _compute_w(0)
    pltpu.async_copy(x_hbm.at[isrc.at[0]], rows.at[0], sem_g[0])
    for b6 in range(6):
        _chunk(0, b6, wait_sc=(b6 >= 1))

    def _main(g, carry):
        for b6 in range(6):
            _chunk(g, b6)
        return carry

    lax.fori_loop(1, NCH // 6, _main, 0)
    for b6 in range(5):
        q = 120 + b6
        _chunk(20, b6, has_p1=(q + 1 < NCH), has_p2=(q + 2 < NCH))

    _wait_rows(0, sem_s[1])
    _wait_rows(1, sem_s[0])

    plsc.subcore_barrier()

    descs = []
    for j in range((NZC + NS - 1) // NS):
        k = s + NS * j
        r0 = pl.multiple_of(k * ZR, 8)
        if (j + 1) * NS <= NZC:
            descs.append(pltpu.async_copy(
                p_acc.at[pl.ds(r0, ZR)], pp_hbm.at[c, pl.ds(r0, ZR)], sem_g[0]))
        else:
            @pl.when(k < NZC)
            def _():
                pltpu.sync_copy(p_acc.at[pl.ds(r0, ZR)],
                                pp_hbm.at[c, pl.ds(r0, ZR)])
    for d in descs:
        d.wait()


def _make_sc_scatter():
    return pl.kernel(
        _sc_body,
    out_type=(
        jax.ShapeDtypeStruct((NC, N, D), jnp.float32),
        jax.ShapeDtypeStruct((N,), jnp.float32),
    ),
    mesh=plsc.VectorSubcoreMesh(core_axis_name="c", subcore_axis_name="s",
                                num_cores=NC, num_subcores=NS),
    compiler_params=pltpu.CompilerParams(needs_layout_passes=False),
    scratch_types=[
        pltpu.VMEM_SHARED((N, D), jnp.float32),
        pltpu.VMEM_SHARED((N,), jnp.float32),
        pltpu.VMEM((N,), jnp.float32),
        pltpu.VMEM((3, CH, D), jnp.float32),
        pltpu.VMEM((6, CH), jnp.int32),
        pltpu.VMEM((6, CH), jnp.int32),
        pltpu.VMEM((6, CH), jnp.float32),
        pltpu.VMEM((6 * CH,), jnp.float32),
        pltpu.VMEM((KD, CH), jnp.int32),
        pltpu.VMEM((KD, CH), jnp.float32),
        pltpu.SemaphoreType.DMA,
        pltpu.SemaphoreType.DMA,
        pltpu.SemaphoreType.DMA,
        pltpu.SemaphoreType.DMA,
        pltpu.SemaphoreType.DMA,
        pltpu.SemaphoreType.DMA,
    ],
)


BLK = 1000
NB = N // BLK


def _tc_body(pa_ref, pb_ref, dinv_ref, x_ref, h_ref, batch_ref,
             wcz_ref, bcz_ref, wcr_ref, bcr_ref, wch_ref, bch_ref,
             wlz_ref, blz_ref, wlr_ref, blr_ref, wlh_ref, blh_ref,
             wout_ref, bout_ref, hn_ref, out_ref, sums_ref, cnt_ref):
    i = pl.program_id(0)

    @pl.when(i == 0)
    def _():
        sums_ref[...] = jnp.zeros_like(sums_ref)
        cnt_ref[...] = jnp.zeros_like(cnt_ref)

    dv = dinv_ref[...]
    x = x_ref[...]
    h = h_ref[...]
    p = pa_ref[...] + pb_ref[...] + (dv * dv) * x

    f32 = jnp.float32
    dot = functools.partial(jnp.dot, preferred_element_type=f32)
    cz = dot(p, wcz_ref[...]) + bcz_ref[...]
    cr = dot(p, wcr_ref[...]) + bcr_ref[...]
    ch = dot(p, wch_ref[...]) + bch_ref[...]

    wlz = wlz_ref[...]
    wlr = wlr_ref[...]
    wlh = wlh_ref[...]
    z = jax.nn.sigmoid(dot(cz, wlz[:H]) + dot(h, wlz[H:]) + blz_ref[...])
    r = jax.nn.sigmoid(dot(cr, wlr[:H]) + dot(h, wlr[H:]) + blr_ref[...])
    ht = jnp.tanh(dot(ch, wlh[:H]) + dot(h * r, wlh[H:]) + blh_ref[...])
    hn = z * h + (1.0 - z) * ht
    hn_ref[...] = hn

    b2 = batch_ref[0]
    gids = lax.broadcasted_iota(jnp.int32, (G, BLK), 0)
    m = (gids == b2).astype(f32)
    sums_ref[...] += dot(m, hn)
    cnt_ref[...] += jnp.sum(m, axis=1)[:, None]

    @pl.when(i == NB - 1)
    def _():
        pooled = sums_ref[...] / jnp.maximum(cnt_ref[...], 1.0)
        out_ref[...] = dot(pooled, wout_ref[...]) + bout_ref[...]


def _tc_call(pa, pb, dinv2d, x, h, batch3d, *weights):
    full = lambda shape: pl.BlockSpec(shape, lambda i: tuple(0 for _ in shape))
    wspecs = []
    for wgt in weights:
        wspecs.append(full(wgt.shape))
    return pl.pallas_call(
        _tc_body,
        grid=(NB,),
        in_specs=[
            pl.BlockSpec((BLK, D), lambda i: (i, 0)),
            pl.BlockSpec((BLK, D), lambda i: (i, 0)),
            pl.BlockSpec((BLK, 1), lambda i: (i, 0)),
            pl.BlockSpec((BLK, D), lambda i: (i, 0)),
            pl.BlockSpec((BLK, H), lambda i: (i, 0)),
            pl.BlockSpec((1, 1, BLK), lambda i: (i, 0, 0)),
        ] + wspecs,
        out_specs=[
            pl.BlockSpec((BLK, H), lambda i: (i, 0)),
            pl.BlockSpec((G, H), lambda i: (0, 0)),
        ],
        out_shape=[
            jax.ShapeDtypeStruct((N, H), jnp.float32),
            jax.ShapeDtypeStruct((G, H), jnp.float32),
        ],
        scratch_shapes=[
            pltpu.VMEM((G, H), jnp.float32),
            pltpu.VMEM((G, H), jnp.float32),
        ],
    )(pa, pb, dinv2d, x, h, batch3d, *weights)


@jax.jit
def kernel(x, edge_index, edge_attr, batch, hidden_state, Wcz, bcz, Wcr, bcr,
           Wch, bch, Wlz, blz, Wlr, blr, Wlh, blh, Wout, bout):
    src = edge_index[0]
    dst = edge_index[1]
    pp, dinv = _make_sc_scatter()(src, dst, edge_attr, x)
    hn, out = _tc_call(
        pp[0], pp[1], dinv.reshape(N, 1), x, hidden_state,
        batch.reshape(NB, 1, BLK),
        Wcz, bcz.reshape(1, H), Wcr, bcr.reshape(1, H), Wch, bch.reshape(1, H),
        Wlz, blz.reshape(1, H), Wlr, blr.reshape(1, H), Wlh, blh.reshape(1, H),
        Wout, bout.reshape(1, H),
    )
    return out, hn


# speedup vs baseline: 1.0216x; 1.0216x over previous
import functools

import jax
import jax.numpy as jnp
from jax import lax
from jax.experimental import pallas as pl
from jax.experimental.pallas import tpu as pltpu
from jax.experimental.pallas import tpu_sc as plsc

N = 10000
E = 320000
D = 128
H = 128
G = 64

NC = 2
NS = 16
NW = NC * NS
L = 16

CH = 80
EPT = E // NW
EPS = E // NS
ZR = 80
NZC = N // ZR
KD = 10


def _rsqrt16(y):
    i = lax.bitcast_convert_type(y, jnp.int32)
    i = jnp.int32(0x5F3759DF) - (i >> 1)
    r = lax.bitcast_convert_type(i, jnp.float32)
    half = y * 0.5
    for _ in range(3):
        r = r * (1.5 - half * r * r)
    return r


def _sc_body(src_hbm, dst_hbm, ew_hbm, x_hbm, pp_hbm, dinv_hbm,
             p_acc, deg_acc, dinvv, rows,
             isrc, idst, ewb, wbuf, idstd, ewbd,
             sl0, sl1, sg0, sg1, ss0, ss1):
    sem_l = (sl0, sl1)
    sem_g = (sg0, sg1)
    sem_s = (ss0, ss1)
    c = lax.axis_index("c")
    s = lax.axis_index("s")
    wid = c * NS + s

    zv = jnp.zeros((L,), jnp.float32)

    def _zero_rows(k, carry):
        rows[0, k // 8, pl.ds((k % 8) * L, L)] = zv
        return carry

    lax.fori_loop(0, ZR * 8, _zero_rows, 0)
    for j in range(CH // L):
        ewbd[0, pl.ds(j * L, L)] = zv

    descs = []
    for j in range((NZC + NS - 1) // NS):
        k = s + NS * j
        r0 = pl.multiple_of(k * ZR, 8)
        if (j + 1) * NS <= NZC:
            descs.append(pltpu.async_copy(
                rows.at[0], p_acc.at[pl.ds(r0, ZR)], sem_g[0]))
            descs.append(pltpu.async_copy(
                ewbd.at[0], deg_acc.at[pl.ds(r0, ZR)], sem_l[0]))
        else:
            @pl.when(k < NZC)
            def _():
                pltpu.sync_copy(rows.at[0], p_acc.at[pl.ds(r0, ZR)])
                pltpu.sync_copy(ewbd.at[0], deg_acc.at[pl.ds(r0, ZR)])
    for d in descs:
        d.wait()

    plsc.subcore_barrier()

    def _deg_group(g, carry):
        base = pl.multiple_of(s * EPS + g * (KD * CH), 8)
        descs = []
        for k in range(KD):
            b = base + k * CH
            descs.append(pltpu.async_copy(
                dst_hbm.at[pl.ds(b, CH)], idstd.at[k], sem_l[0]))
            descs.append(pltpu.async_copy(
                ew_hbm.at[pl.ds(b, CH)], ewbd.at[k], sem_l[0]))
        for d in descs:
            d.wait()
        descs = []
        for k in range(KD):
            descs.append(pltpu.async_copy(
                ewbd.at[k], deg_acc.at[idstd.at[k]], sem_s[0], add=True))
        for d in descs:
            d.wait()
        return carry

    lax.fori_loop(0, EPS // (KD * CH), _deg_group, 0)

    plsc.subcore_barrier()

    pltpu.sync_copy(deg_acc, dinvv)

    def _dinv(j, carry):
        y = dinvv[pl.ds(j * L, L)] + 1.0
        dinvv[pl.ds(j * L, L)] = _rsqrt16(y)
        return carry

    lax.fori_loop(0, N // L, _dinv, 0)

    @pl.when(jnp.logical_and(c == 0, s == 0))
    def _():
        pltpu.sync_copy(dinvv, dinv_hbm)

    NCH = EPT // CH

    def _q_base(q):
        return pl.multiple_of(wid * EPT + q * CH, 8)

    def _issue_loads(q, b6, par):
        b = _q_base(q)
        pltpu.async_copy(src_hbm.at[pl.ds(b, CH)], isrc.at[b6], sem_l[par])
        pltpu.async_copy(dst_hbm.at[pl.ds(b, CH)], idst.at[b6], sem_l[par])
        pltpu.async_copy(ew_hbm.at[pl.ds(b, CH)], ewb.at[b6], sem_l[par])

    def _wait_loads(b6, par):
        z = pl.ds(0, CH)
        pltpu.make_async_copy(src_hbm.at[z], isrc.at[b6], sem_l[par]).wait()
        pltpu.make_async_copy(dst_hbm.at[z], idst.at[b6], sem_l[par]).wait()
        pltpu.make_async_copy(ew_hbm.at[z], ewb.at[b6], sem_l[par]).wait()

    def _compute_w(b6):
        for j in range(CH // L):
            sv = isrc[b6, pl.ds(j * L, L)]
            dv = idst[b6, pl.ds(j * L, L)]
            dis = plsc.load_gather(dinvv, [sv])
            did = plsc.load_gather(dinvv, [dv])
            wbuf[pl.ds(b6 * CH + j * L, L)] = (
                ewb[b6, pl.ds(j * L, L)] * dis * did)

    def _wait_rows(b3, sem):
        pltpu.make_async_copy(x_hbm.at[pl.ds(0, CH)], rows.at[b3], sem).wait()

    def _chunk(g, b6, has_p1=True, has_p2=True, wait_sc=True):
        q1b6, q2b6 = (b6 + 1) % 6, (b6 + 2) % 6
        b3, q1b3 = b6 % 3, (b6 + 1) % 3
        par, q1par = b6 % 2, (b6 + 1) % 2
        if has_p1:
            _wait_loads(q1b6, q1par)
            _compute_w(q1b6)
            if wait_sc:
                _wait_rows((b6 + 2) % 3, sem_s[q1par])
            pltpu.async_copy(x_hbm.at[isrc.at[q1b6]], rows.at[q1b3],
                             sem_g[q1par])
        if has_p2:
            _issue_loads(6 * g + b6 + 2, q2b6, par)
        _wait_rows(b3, sem_g[par])

        @plsc.parallel_loop(b6 * CH, (b6 + 1) * CH, 1, unroll=4)
        def _scale(e):
            wk = plsc.load_gather(wbuf, [jnp.full((L,), e, jnp.int32)])
            for jj in range(D // L):
                rows[b3, e - b6 * CH, pl.ds(jj * L, L)] = (
                    rows[b3, e - b6 * CH, pl.ds(jj * L, L)] * wk)

        pltpu.async_copy(rows.at[b3], p_acc.at[idst.at[b6]], sem_s[par],
                         add=True)

    _issue_loads(0, 0, 0)
    _issue_loads(1, 1, 1)
    _wait_loads(0, 0)
    _compute_w(0)
    pltpu.async_copy(x_hbm.at[isrc.at[0]], rows.at[0], sem_g[0])
    for b6 in range(6):
        _chunk(0, b6, wait_sc=(b6 >= 1))

    def _main(g, carry):
        for b6 in range(6):
            _chunk(g, b6)
        return carry

    lax.fori_loop(1, NCH // 6, _main, 0)
    for b6 in range(5):
        q = 120 + b6
        _chunk(20, b6, has_p1=(q + 1 < NCH), has_p2=(q + 2 < NCH))

    _wait_rows(0, sem_s[1])
    _wait_rows(1, sem_s[0])

    plsc.subcore_barrier()

    descs = []
    for j in range((NZC + NS - 1) // NS):
        k = s + NS * j
        r0 = pl.multiple_of(k * ZR, 8)
        if (j + 1) * NS <= NZC:
            descs.append(pltpu.async_copy(
                p_acc.at[pl.ds(r0, ZR)], pp_hbm.at[c, pl.ds(r0, ZR)], sem_g[0]))
        else:
            @pl.when(k < NZC)
            def _():
                pltpu.sync_copy(p_acc.at[pl.ds(r0, ZR)],
                                pp_hbm.at[c, pl.ds(r0, ZR)])
    for d in descs:
        d.wait()


def _make_sc_scatter():
    return pl.kernel(
        _sc_body,
    out_type=(
        jax.ShapeDtypeStruct((NC, N, D), jnp.float32),
        jax.ShapeDtypeStruct((N,), jnp.float32),
    ),
    mesh=plsc.VectorSubcoreMesh(core_axis_name="c", subcore_axis_name="s",
                                num_cores=NC, num_subcores=NS),
    compiler_params=pltpu.CompilerParams(needs_layout_passes=False),
    scratch_types=[
        pltpu.VMEM_SHARED((N, D), jnp.float32),
        pltpu.VMEM_SHARED((N,), jnp.float32),
        pltpu.VMEM((N,), jnp.float32),
        pltpu.VMEM((3, CH, D), jnp.float32),
        pltpu.VMEM((6, CH), jnp.int32),
        pltpu.VMEM((6, CH), jnp.int32),
        pltpu.VMEM((6, CH), jnp.float32),
        pltpu.VMEM((6 * CH,), jnp.float32),
        pltpu.VMEM((KD, CH), jnp.int32),
        pltpu.VMEM((KD, CH), jnp.float32),
        pltpu.SemaphoreType.DMA,
        pltpu.SemaphoreType.DMA,
        pltpu.SemaphoreType.DMA,
        pltpu.SemaphoreType.DMA,
        pltpu.SemaphoreType.DMA,
        pltpu.SemaphoreType.DMA,
    ],
)


BLK = 1000
NB = N // BLK


def _tc_body(pa_ref, pb_ref, dinv_ref, x_ref, h_ref, batch_ref,
             wcz_ref, bcz_ref, wcr_ref, bcr_ref, wch_ref, bch_ref,
             wlz_ref, blz_ref, wlr_ref, blr_ref, wlh_ref, blh_ref,
             wout_ref, bout_ref, hn_ref, out_ref, sums_ref, cnt_ref):
    i = pl.program_id(0)

    @pl.when(i == 0)
    def _():
        sums_ref[...] = jnp.zeros_like(sums_ref)
        cnt_ref[...] = jnp.zeros_like(cnt_ref)

    dv = dinv_ref[...]
    x = x_ref[...]
    h = h_ref[...]
    p = pa_ref[...] + pb_ref[...] + (dv * dv) * x

    f32 = jnp.float32
    dot = functools.partial(jnp.dot, preferred_element_type=f32)
    cz = dot(p, wcz_ref[...]) + bcz_ref[...]
    cr = dot(p, wcr_ref[...]) + bcr_ref[...]
    ch = dot(p, wch_ref[...]) + bch_ref[...]

    wlz = wlz_ref[...]
    wlr = wlr_ref[...]
    wlh = wlh_ref[...]
    z = jax.nn.sigmoid(dot(cz, wlz[:H]) + dot(h, wlz[H:]) + blz_ref[...])
    r = jax.nn.sigmoid(dot(cr, wlr[:H]) + dot(h, wlr[H:]) + blr_ref[...])
    ht = jnp.tanh(dot(ch, wlh[:H]) + dot(h * r, wlh[H:]) + blh_ref[...])
    hn = z * h + (1.0 - z) * ht
    hn_ref[...] = hn

    b2 = batch_ref[0]
    gids = lax.broadcasted_iota(jnp.int32, (G, BLK), 0)
    m = (gids == b2).astype(f32)
    sums_ref[...] += dot(m, hn)
    cnt_ref[...] += jnp.sum(m, axis=1)[:, None]

    @pl.when(i == NB - 1)
    def _():
        pooled = sums_ref[...] / jnp.maximum(cnt_ref[...], 1.0)
        out_ref[...] = dot(pooled, wout_ref[...]) + bout_ref[...]


def _tc_call(pa, pb, dinv2d, x, h, batch3d, *weights):
    full = lambda shape: pl.BlockSpec(shape, lambda i: tuple(0 for _ in shape))
    wspecs = []
    for wgt in weights:
        wspecs.append(full(wgt.shape))
    return pl.pallas_call(
        _tc_body,
        grid=(NB,),
        in_specs=[
            pl.BlockSpec((BLK, D), lambda i: (i, 0)),
            pl.BlockSpec((BLK, D), lambda i: (i, 0)),
            pl.BlockSpec((BLK, 1), lambda i: (i, 0)),
            pl.BlockSpec((BLK, D), lambda i: (i, 0)),
            pl.BlockSpec((BLK, H), lambda i: (i, 0)),
            pl.BlockSpec((1, 1, BLK), lambda i: (i, 0, 0)),
        ] + wspecs,
        out_specs=[
            pl.BlockSpec((BLK, H), lambda i: (i, 0)),
            pl.BlockSpec((G, H), lambda i: (0, 0)),
        ],
        out_shape=[
            jax.ShapeDtypeStruct((N, H), jnp.float32),
            jax.ShapeDtypeStruct((G, H), jnp.float32),
        ],
        scratch_shapes=[
            pltpu.VMEM((G, H), jnp.float32),
            pltpu.VMEM((G, H), jnp.float32),
        ],
    )(pa, pb, dinv2d, x, h, batch3d, *weights)


@jax.jit
def kernel(x, edge_index, edge_attr, batch, hidden_state, Wcz, bcz, Wcr, bcr,
           Wch, bch, Wlz, blz, Wlr, blr, Wlh, blh, Wout, bout):
    src = edge_index[0]
    dst = edge_index[1]
    pp, dinv = _make_sc_scatter()(src, dst, edge_attr, x)
    hn, out = _tc_call(
        pp[0], pp[1], dinv.reshape(N, 1), x, hidden_state,
        batch.reshape(NB, 1, BLK),
        Wcz, bcz.reshape(1, H), Wcr, bcr.reshape(1, H), Wch, bch.reshape(1, H),
        Wlz, blz.reshape(1, H), Wlr, blr.reshape(1, H), Wlh, blh.reshape(1, H),
        Wout, bout.reshape(1, H),
    )
    return out, hn
